# baseline (device time: 126701 ns/iter reference)
import jax
import jax.numpy as jnp
from jax import lax
from jax.experimental import pallas as pl
from jax.experimental.pallas import tpu as pltpu

N_DEV = 16
B = 2
S_LOC = 128
HQ = 4
DH = 64
S_KV = N_DEV * S_LOC
BLK = 64
ROWS = B * HQ * S_LOC


def kernel(x, Wq, K_ext, V_ext, Wo):
    d_model = x.shape[-1]

    kt = jnp.transpose(K_ext, (0, 2, 1, 3)).reshape(ROWS, DH)
    vt = jnp.transpose(V_ext, (0, 2, 1, 3)).reshape(ROWS, DH)
    kv = jnp.concatenate([kt, vt], axis=0).astype(jnp.bfloat16)

    def body(x_ref, wq_ref, kv_ref, wo_ref, out_ref,
             kvf, local_sem, send_sems, recv_sems):
        me = lax.axis_index("i")
        left = lax.rem(me + N_DEV - 1, N_DEV)
        right = lax.rem(me + 1, N_DEV)

        copy = pltpu.make_async_copy(kv_ref, kvf.at[me], local_sem)
        copy.start()
        copy.wait()

        barrier = pltpu.get_barrier_semaphore()
        for nbr in (left, right):
            pl.semaphore_signal(barrier, inc=1, device_id=(nbr,),
                                device_id_type=pl.DeviceIdType.MESH)
        pl.semaphore_wait(barrier, 2)

        for h in range(N_DEV - 1):
            src_o = lax.rem(me + N_DEV - h, N_DEV)
            rdma = pltpu.make_async_remote_copy(
                src_ref=kvf.at[src_o],
                dst_ref=kvf.at[src_o],
                send_sem=send_sems.at[h],
                recv_sem=recv_sems.at[h],
                device_id=(right,),
                device_id_type=pl.DeviceIdType.MESH,
            )
            rdma.start()
            rdma.wait()

        r = lax.broadcasted_iota(jnp.int32, (S_LOC, S_KV), 0)
        c = lax.broadcasted_iota(jnp.int32, (S_LOC, S_KV), 1)
        qblk = (me * S_LOC + r) // BLK
        kblk = c // BLK
        mask = (qblk == kblk) | (kblk == 0) | (lax.rem(qblk + kblk, 3) == 0)

        for b in range(B):
            xb = x_ref[b].astype(jnp.bfloat16)
            q_b = jnp.dot(xb, wq_ref[:].astype(jnp.bfloat16),
                          preferred_element_type=jnp.float32)
            ctx_heads = []
            for h in range(HQ):
                r0 = (b * HQ + h) * S_LOC
                k_bh = jnp.concatenate(
                    [kvf[j, r0:r0 + S_LOC, :] for j in range(N_DEV)], axis=0)
                v_bh = jnp.concatenate(
                    [kvf[j, ROWS + r0:ROWS + r0 + S_LOC, :]
                     for j in range(N_DEV)], axis=0)
                q_bh = q_b[:, h * DH:(h + 1) * DH].astype(jnp.bfloat16)
                scores = lax.dot_general(
                    q_bh, k_bh, (((1,), (1,)), ((), ())),
                    preferred_element_type=jnp.float32) * 0.125
                scores = jnp.where(mask, scores, -1e9)
                m = jnp.max(scores, axis=1, keepdims=True)
                w = jnp.exp(scores - m)
                w = w / jnp.sum(w, axis=1, keepdims=True)
                ctx = jnp.dot(w.astype(jnp.bfloat16), v_bh,
                              preferred_element_type=jnp.float32)
                ctx_heads.append(ctx)
            ctx_b = jnp.concatenate(ctx_heads, axis=1).astype(jnp.bfloat16)
            out_ref[b] = jnp.dot(ctx_b, wo_ref[:].astype(jnp.bfloat16),
                                 preferred_element_type=jnp.float32)

    return pl.pallas_call(
        body,
        out_shape=jax.ShapeDtypeStruct((B, S_LOC, d_model), jnp.float32),
        in_specs=[pl.BlockSpec(memory_space=pltpu.VMEM)] * 4,
        out_specs=pl.BlockSpec(memory_space=pltpu.VMEM),
        scratch_shapes=[
            pltpu.VMEM((N_DEV, 2 * ROWS, DH), jnp.bfloat16),
            pltpu.SemaphoreType.DMA,
            pltpu.SemaphoreType.DMA((N_DEV - 1,)),
            pltpu.SemaphoreType.DMA((N_DEV - 1,)),
        ],
        compiler_params=pltpu.CompilerParams(collective_id=0),
    )(x, Wq, kv, Wo)


# device time: 121943 ns/iter; 1.0390x vs baseline; 1.0390x over previous
import jax
import jax.numpy as jnp
from jax import lax
from jax.experimental import pallas as pl
from jax.experimental.pallas import tpu as pltpu

N_DEV = 16
B = 2
S_LOC = 128
HQ = 4
DH = 64
BLK = 64
N_BH = B * HQ
ROWS = N_BH * S_LOC


def kernel(x, Wq, K_ext, V_ext, Wo):
    d_model = x.shape[-1]

    kt = jnp.transpose(K_ext, (0, 2, 1, 3)).reshape(ROWS, DH)
    vt = jnp.transpose(V_ext, (0, 2, 1, 3)).reshape(ROWS, DH)
    kv = jnp.concatenate([kt, vt], axis=0).astype(jnp.bfloat16)

    def body(x_ref, wq_ref, kv_ref, wo_ref, out_ref,
             kvf, local_sem, send_sems, recv_sems):
        me = lax.axis_index("i")
        left = lax.rem(me + N_DEV - 1, N_DEV)
        right = lax.rem(me + 1, N_DEV)

        copy = pltpu.make_async_copy(kv_ref, kvf.at[me], local_sem)
        copy.start()
        copy.wait()

        barrier = pltpu.get_barrier_semaphore()
        for nbr in (left, right):
            pl.semaphore_signal(barrier, inc=1, device_id=(nbr,),
                                device_id_type=pl.DeviceIdType.MESH)
        pl.semaphore_wait(barrier, 2)

        q_heads = []
        for b in range(B):
            q_b = jnp.dot(x_ref[b].astype(jnp.bfloat16),
                          wq_ref[:].astype(jnp.bfloat16),
                          preferred_element_type=jnp.float32)
            for h in range(HQ):
                q_heads.append(q_b[:, h * DH:(h + 1) * DH].astype(jnp.bfloat16))

        r_sub = lax.broadcasted_iota(jnp.int32, (S_LOC, S_LOC), 0)
        c_sub = lax.broadcasted_iota(jnp.int32, (S_LOC, S_LOC), 1)
        qblk = 2 * me + r_sub // BLK

        m_s = [None] * N_BH
        l_s = [None] * N_BH
        acc_s = [None] * N_BH
        rdmas = []

        for step in range(N_DEV):
            o = lax.rem(me + N_DEV - step, N_DEV)
            if step < N_DEV - 1:
                rdma = pltpu.make_async_remote_copy(
                    src_ref=kvf.at[o],
                    dst_ref=kvf.at[o],
                    send_sem=send_sems.at[step],
                    recv_sem=recv_sems.at[step],
                    device_id=(right,),
                    device_id_type=pl.DeviceIdType.MESH,
                )
                rdma.start()
                rdmas.append(rdma)

            kblk = 2 * o + c_sub // BLK
            mask = (qblk == kblk) | (kblk == 0) | (lax.rem(qblk + kblk, 3) == 0)
            for idx in range(N_BH):
                r0 = idx * S_LOC
                k_j = kvf[o, r0:r0 + S_LOC, :]
                v_j = kvf[o, ROWS + r0:ROWS + r0 + S_LOC, :]
                s = lax.dot_general(
                    q_heads[idx], k_j, (((1,), (1,)), ((), ())),
                    preferred_element_type=jnp.float32) * 0.125
                s = jnp.where(mask, s, -1e9)
                if step == 0:
                    m = jnp.max(s, axis=1, keepdims=True)
                    w = jnp.exp(s - m)
                    l = jnp.sum(w, axis=1, keepdims=True)
                    acc = jnp.dot(w.astype(jnp.bfloat16), v_j,
                                  preferred_element_type=jnp.float32)
                else:
                    m = jnp.maximum(m_s[idx], jnp.max(s, axis=1, keepdims=True))
                    alpha = jnp.exp(m_s[idx] - m)
                    w = jnp.exp(s - m)
                    l = l_s[idx] * alpha + jnp.sum(w, axis=1, keepdims=True)
                    acc = acc_s[idx] * alpha + jnp.dot(
                        w.astype(jnp.bfloat16), v_j,
                        preferred_element_type=jnp.float32)
                m_s[idx], l_s[idx], acc_s[idx] = m, l, acc

            if step < N_DEV - 1:
                rdma.wait_recv()

        for rdma in rdmas:
            rdma.wait_send()

        for b in range(B):
            ctx_b = jnp.concatenate(
                [acc_s[b * HQ + h] / l_s[b * HQ + h] for h in range(HQ)],
                axis=1).astype(jnp.bfloat16)
            out_ref[b] = jnp.dot(ctx_b, wo_ref[:].astype(jnp.bfloat16),
                                 preferred_element_type=jnp.float32)

    return pl.pallas_call(
        body,
        out_shape=jax.ShapeDtypeStruct((B, S_LOC, d_model), jnp.float32),
        in_specs=[pl.BlockSpec(memory_space=pltpu.VMEM)] * 4,
        out_specs=pl.BlockSpec(memory_space=pltpu.VMEM),
        scratch_shapes=[
            pltpu.VMEM((N_DEV, 2 * ROWS, DH), jnp.bfloat16),
            pltpu.SemaphoreType.DMA,
            pltpu.SemaphoreType.DMA((N_DEV - 1,)),
            pltpu.SemaphoreType.DMA((N_DEV - 1,)),
        ],
        compiler_params=pltpu.CompilerParams(collective_id=0),
    )(x, Wq, kv, Wo)


# device time: 103875 ns/iter; 1.2197x vs baseline; 1.1739x over previous
import jax
import jax.numpy as jnp
from jax import lax
from jax.experimental import pallas as pl
from jax.experimental.pallas import tpu as pltpu

N_DEV = 16
B = 2
S_LOC = 128
HQ = 4
DH = 64
BLK = 64
N_BH = B * HQ
ROWS = N_BH * S_LOC
NEG = -1e9


def kernel(x, Wq, K_ext, V_ext, Wo):
    d_model = x.shape[-1]

    kt = jnp.transpose(K_ext, (0, 2, 1, 3)).reshape(ROWS, DH)
    vt = jnp.transpose(V_ext, (0, 2, 1, 3)).reshape(ROWS, DH)
    kv = jnp.concatenate([kt, vt], axis=0).astype(jnp.bfloat16)

    def body(x_ref, wq_ref, kv_ref, wo_ref, out_ref, kvf, local_sem,
             p1us, p1ur, p1ds, p1dr, p2s, p2r, p3s, p3r):
        me = lax.axis_index("i")
        z = me // 4
        p = lax.rem(me, 4)
        up = lax.rem(me + 4, N_DEV)
        dn = lax.rem(me + N_DEV - 4, N_DEV)
        py = 4 * z + (3 - p)
        px = 4 * z + jnp.bitwise_xor(p, 1)

        copy = pltpu.make_async_copy(kv_ref, kvf.at[me], local_sem)
        copy.start()
        copy.wait()

        barrier = pltpu.get_barrier_semaphore()
        for nbr in (up, dn, py, px):
            pl.semaphore_signal(barrier, inc=1, device_id=(nbr,),
                                device_id_type=pl.DeviceIdType.MESH)
        pl.semaphore_wait(barrier, 4)

        deferred = []

        def remote(slot, ssem, rsem, target):
            return pltpu.make_async_remote_copy(
                src_ref=kvf.at[slot], dst_ref=kvf.at[slot],
                send_sem=ssem, recv_sem=rsem,
                device_id=(target,), device_id_type=pl.DeviceIdType.MESH)

        q_heads = [None] * N_BH
        m_s = [None] * N_BH
        l_s = [None] * N_BH
        acc_s = [None] * N_BH

        r_sub = lax.broadcasted_iota(jnp.int32, (S_LOC, S_LOC), 0)
        c_sub = lax.broadcasted_iota(jnp.int32, (S_LOC, S_LOC), 1)

        def compute_q():
            for b in range(B):
                q_b = jnp.dot(x_ref[b].astype(jnp.bfloat16),
                              wq_ref[:].astype(jnp.bfloat16),
                              preferred_element_type=jnp.float32)
                for h in range(HQ):
                    idx = b * HQ + h
                    q_heads[idx] = q_b[:, h * DH:(h + 1) * DH].astype(
                        jnp.bfloat16)
                    m_s[idx] = jnp.full((S_LOC, 1), -1e30, jnp.float32)
                    l_s[idx] = jnp.zeros((S_LOC, 1), jnp.float32)
                    acc_s[idx] = jnp.zeros((S_LOC, DH), jnp.float32)

        qblk_base = 2 * me + r_sub // BLK

        def consume(o):
            kblk = 2 * o + c_sub // BLK
            mask = ((qblk_base == kblk) | (kblk == 0)
                    | (lax.rem(qblk_base + kblk, 3) == 0))
            for idx in range(N_BH):
                r0 = idx * S_LOC
                k_j = kvf[o, r0:r0 + S_LOC, :]
                v_j = kvf[o, ROWS + r0:ROWS + r0 + S_LOC, :]
                s = lax.dot_general(
                    q_heads[idx], k_j, (((1,), (1,)), ((), ())),
                    preferred_element_type=jnp.float32) * 0.125
                s = jnp.where(mask, s, NEG)
                m = jnp.maximum(m_s[idx], jnp.max(s, axis=1, keepdims=True))
                alpha = jnp.exp(m_s[idx] - m)
                w = jnp.exp(s - m)
                l = l_s[idx] * alpha + jnp.sum(w, axis=1, keepdims=True)
                acc = acc_s[idx] * alpha + jnp.dot(
                    w.astype(jnp.bfloat16), v_j,
                    preferred_element_type=jnp.float32)
                m_s[idx], l_s[idx], acc_s[idx] = m, l, acc

        for s in range(3):
            up_cond = (z <= 2) & (z - s >= 0)
            dn_cond = (z >= 1) & (z + s <= 3)

            @pl.when(up_cond)
            def _(s=s):
                remote(4 * (z - s) + p, p1us.at[s], p1ur.at[s], up).start()

            @pl.when(dn_cond)
            def _(s=s):
                remote(4 * (z + s) + p, p1ds.at[s], p1dr.at[s], dn).start()

            deferred.append((up_cond,
                             lambda s=s: remote(4 * (z - s) + p, p1us.at[s],
                                                p1ur.at[s], up)))
            deferred.append((dn_cond,
                             lambda s=s: remote(4 * (z + s) + p, p1ds.at[s],
                                                p1dr.at[s], dn)))

            if s == 0:
                compute_q()

            rb_cond = (z >= 1) & (z - 1 - s >= 0)
            ra_cond = (z <= 2) & (z + 1 + s <= 3)

            @pl.when(rb_cond)
            def _(s=s):
                remote(4 * (z - 1 - s) + p, p1us.at[s], p1ur.at[s],
                       up).wait_recv()

            @pl.when(ra_cond)
            def _(s=s):
                remote(4 * (z + 1 + s) + p, p1ds.at[s], p1dr.at[s],
                       dn).wait_recv()

        for j in range(4):
            remote(4 * j + p, p2s.at[j], p2r.at[j], py).start()
            deferred.append((None,
                             lambda j=j: remote(4 * j + p, p2s.at[j],
                                                p2r.at[j], py)))

        for j in range(4):
            consume(4 * j + p)

        for j in range(4):
            remote(4 * j + (3 - p), p2s.at[j], p2r.at[j], py).wait_recv()

        for jj in range(8):
            zc, half = jj // 2, jj % 2
            q_col = p if half == 0 else (3 - p)
            remote(4 * zc + q_col, p3s.at[jj], p3r.at[jj], px).start()
            deferred.append((None,
                             lambda jj=jj, zc=zc, half=half: remote(
                                 4 * zc + (p if half == 0 else (3 - p)),
                                 p3s.at[jj], p3r.at[jj], px)))

        for j in range(4):
            consume(4 * j + (3 - p))

        pxp = jnp.bitwise_xor(p, 1)
        for jj in range(8):
            zc, half = jj // 2, jj % 2
            q_col = pxp if half == 0 else (3 - pxp)
            remote(4 * zc + q_col, p3s.at[jj], p3r.at[jj], px).wait_recv()
            consume(4 * zc + q_col)

        for cond, mk in deferred:
            if cond is None:
                mk().wait_send()
            else:
                @pl.when(cond)
                def _(mk=mk):
                    mk().wait_send()

        for b in range(B):
            ctx_b = jnp.concatenate(
                [acc_s[b * HQ + h] / l_s[b * HQ + h] for h in range(HQ)],
                axis=1).astype(jnp.bfloat16)
            out_ref[b] = jnp.dot(ctx_b, wo_ref[:].astype(jnp.bfloat16),
                                 preferred_element_type=jnp.float32)

    return pl.pallas_call(
        body,
        out_shape=jax.ShapeDtypeStruct((B, S_LOC, d_model), jnp.float32),
        in_specs=[pl.BlockSpec(memory_space=pltpu.VMEM)] * 4,
        out_specs=pl.BlockSpec(memory_space=pltpu.VMEM),
        scratch_shapes=[
            pltpu.VMEM((N_DEV, 2 * ROWS, DH), jnp.bfloat16),
            pltpu.SemaphoreType.DMA,
            pltpu.SemaphoreType.DMA((3,)),
            pltpu.SemaphoreType.DMA((3,)),
            pltpu.SemaphoreType.DMA((3,)),
            pltpu.SemaphoreType.DMA((3,)),
            pltpu.SemaphoreType.DMA((4,)),
            pltpu.SemaphoreType.DMA((4,)),
            pltpu.SemaphoreType.DMA((8,)),
            pltpu.SemaphoreType.DMA((8,)),
        ],
        compiler_params=pltpu.CompilerParams(collective_id=0),
    )(x, Wq, kv, Wo)


# device time: 77904 ns/iter; 1.6264x vs baseline; 1.3334x over previous
import jax
import jax.numpy as jnp
from jax import lax
from jax.experimental import pallas as pl
from jax.experimental.pallas import tpu as pltpu

N_DEV = 16
B = 2
S_LOC = 128
HQ = 4
DH = 64
BLK = 64
N_BH = B * HQ
ROWS = N_BH * S_LOC
NEG = -1e9


def kernel(x, Wq, K_ext, V_ext, Wo):
    d_model = x.shape[-1]

    kt = jnp.transpose(K_ext, (0, 2, 1, 3)).reshape(ROWS, DH)
    vt = jnp.transpose(V_ext, (0, 2, 1, 3)).reshape(ROWS, DH)
    kv = jnp.stack([kt, vt], axis=0).astype(jnp.bfloat16)

    def body(x_ref, wq_ref, kv_ref, wo_ref, out_ref, kvf, local_sem,
             p1us, p1ur, p1ds, p1dr, p2ys, p2yr, p2xs, p2xr,
             p3xs, p3xr, p3ys, p3yr):
        me = lax.axis_index("i")
        z = me // 4
        p = lax.rem(me, 4)
        up = lax.rem(me + 4, N_DEV)
        dn = lax.rem(me + N_DEV - 4, N_DEV)
        py = 4 * z + (3 - p)
        px = 4 * z + jnp.bitwise_xor(p, 1)

        copy = pltpu.make_async_copy(kv_ref, kvf.at[me], local_sem)
        copy.start()
        copy.wait()

        barrier = pltpu.get_barrier_semaphore()
        for nbr in (up, dn, py, px):
            pl.semaphore_signal(barrier, inc=1, device_id=(nbr,),
                                device_id_type=pl.DeviceIdType.MESH)
        pl.semaphore_wait(barrier, 4)

        deferred = []

        def remote(slot, ssem, rsem, target, half=None):
            ref = kvf.at[slot] if half is None else kvf.at[slot, half]
            return pltpu.make_async_remote_copy(
                src_ref=ref, dst_ref=ref,
                send_sem=ssem, recv_sem=rsem,
                device_id=(target,), device_id_type=pl.DeviceIdType.MESH)

        q_heads = [None] * N_BH
        m_s = [None] * N_BH
        l_s = [None] * N_BH
        acc_s = [None] * N_BH

        r_sub = lax.broadcasted_iota(jnp.int32, (S_LOC, S_LOC), 0)
        c_sub = lax.broadcasted_iota(jnp.int32, (S_LOC, S_LOC), 1)

        def compute_q():
            for b in range(B):
                q_b = jnp.dot(x_ref[b].astype(jnp.bfloat16),
                              wq_ref[:].astype(jnp.bfloat16),
                              preferred_element_type=jnp.float32)
                for h in range(HQ):
                    idx = b * HQ + h
                    q_heads[idx] = q_b[:, h * DH:(h + 1) * DH].astype(
                        jnp.bfloat16)
                    m_s[idx] = jnp.full((S_LOC, 1), -1e30, jnp.float32)
                    l_s[idx] = jnp.zeros((S_LOC, 1), jnp.float32)
                    acc_s[idx] = jnp.zeros((S_LOC, DH), jnp.float32)

        qblk_base = 2 * me + r_sub // BLK

        def consume(o):
            kblk = 2 * o + c_sub // BLK
            mask = ((qblk_base == kblk) | (kblk == 0)
                    | (lax.rem(qblk_base + kblk, 3) == 0))
            for idx in range(N_BH):
                r0 = idx * S_LOC
                k_j = kvf[o, 0, r0:r0 + S_LOC, :]
                v_j = kvf[o, 1, r0:r0 + S_LOC, :]
                s = lax.dot_general(
                    q_heads[idx], k_j, (((1,), (1,)), ((), ())),
                    preferred_element_type=jnp.float32) * 0.125
                s = jnp.where(mask, s, NEG)
                m = jnp.maximum(m_s[idx], jnp.max(s, axis=1, keepdims=True))
                alpha = jnp.exp(m_s[idx] - m)
                w = jnp.exp(s - m)
                l = l_s[idx] * alpha + jnp.sum(w, axis=1, keepdims=True)
                acc = acc_s[idx] * alpha + jnp.dot(
                    w.astype(jnp.bfloat16), v_j,
                    preferred_element_type=jnp.float32)
                m_s[idx], l_s[idx], acc_s[idx] = m, l, acc

        for s in range(3):
            up_cond = (z <= 2) & (z - s >= 0)
            dn_cond = (z >= 1) & (z + s <= 3)

            @pl.when(up_cond)
            def _(s=s):
                remote(4 * (z - s) + p, p1us.at[s], p1ur.at[s], up).start()

            @pl.when(dn_cond)
            def _(s=s):
                remote(4 * (z + s) + p, p1ds.at[s], p1dr.at[s], dn).start()

            deferred.append((up_cond,
                             lambda s=s: remote(4 * (z - s) + p, p1us.at[s],
                                                p1ur.at[s], up)))
            deferred.append((dn_cond,
                             lambda s=s: remote(4 * (z + s) + p, p1ds.at[s],
                                                p1dr.at[s], dn)))

            if s == 0:
                compute_q()

            rb_cond = (z >= 1) & (z - 1 - s >= 0)
            ra_cond = (z <= 2) & (z + 1 + s <= 3)

            @pl.when(rb_cond)
            def _(s=s):
                remote(4 * (z - 1 - s) + p, p1us.at[s], p1ur.at[s],
                       up).wait_recv()

            @pl.when(ra_cond)
            def _(s=s):
                remote(4 * (z + 1 + s) + p, p1ds.at[s], p1dr.at[s],
                       dn).wait_recv()

        pxp = jnp.bitwise_xor(p, 1)
        for j in range(4):
            remote(4 * j + p, p2ys.at[j], p2yr.at[j], py, 0).start()
            remote(4 * j + p, p2xs.at[j], p2xr.at[j], px, 1).start()
            remote(4 * j + p, p3xs.at[j], p3xr.at[j], px, 0).start()
            remote(4 * j + p, p3ys.at[j], p3yr.at[j], py, 1).start()
            deferred.append((None, lambda j=j: remote(4 * j + p, p2ys.at[j],
                                                      p2yr.at[j], py, 0)))
            deferred.append((None, lambda j=j: remote(4 * j + p, p2xs.at[j],
                                                      p2xr.at[j], px, 1)))
            deferred.append((None, lambda j=j: remote(4 * j + p, p3xs.at[j],
                                                      p3xr.at[j], px, 0)))
            deferred.append((None, lambda j=j: remote(4 * j + p, p3ys.at[j],
                                                      p3yr.at[j], py, 1)))

        for j in range(4):
            consume(4 * j + p)

        for j in range(4):
            remote(4 * j + (3 - p), p2ys.at[j], p2yr.at[j], py, 0).wait_recv()
            remote(4 * j + pxp, p2xs.at[j], p2xr.at[j], px, 1).wait_recv()
        for j in range(4):
            remote(4 * j + (3 - p), p3xs.at[4 + j], p3xr.at[4 + j],
                   px, 0).start()
            remote(4 * j + pxp, p3ys.at[4 + j], p3yr.at[4 + j],
                   py, 1).start()
            deferred.append((None, lambda j=j: remote(
                4 * j + (3 - p), p3xs.at[4 + j], p3xr.at[4 + j], px, 0)))
            deferred.append((None, lambda j=j: remote(
                4 * j + pxp, p3ys.at[4 + j], p3yr.at[4 + j], py, 1)))

        for j in range(4):
            remote(4 * j + pxp, p3xs.at[j], p3xr.at[j], px, 0).wait_recv()
            consume(4 * j + pxp)
        for j in range(4):
            remote(4 * j + (3 - p), p3ys.at[j], p3yr.at[j], py, 1).wait_recv()
            consume(4 * j + (3 - p))
        for j in range(4):
            remote(4 * j + (3 - pxp), p3xs.at[4 + j], p3xr.at[4 + j],
                   px, 0).wait_recv()
            remote(4 * j + (3 - pxp), p3ys.at[4 + j], p3yr.at[4 + j],
                   py, 1).wait_recv()
            consume(4 * j + (3 - pxp))

        for cond, mk in deferred:
            if cond is None:
                mk().wait_send()
            else:
                @pl.when(cond)
                def _(mk=mk):
                    mk().wait_send()

        for b in range(B):
            ctx_b = jnp.concatenate(
                [acc_s[b * HQ + h] / l_s[b * HQ + h] for h in range(HQ)],
                axis=1).astype(jnp.bfloat16)
            out_ref[b] = jnp.dot(ctx_b, wo_ref[:].astype(jnp.bfloat16),
                                 preferred_element_type=jnp.float32)

    return pl.pallas_call(
        body,
        out_shape=jax.ShapeDtypeStruct((B, S_LOC, d_model), jnp.float32),
        in_specs=[pl.BlockSpec(memory_space=pltpu.VMEM)] * 4,
        out_specs=pl.BlockSpec(memory_space=pltpu.VMEM),
        scratch_shapes=[
            pltpu.VMEM((N_DEV, 2, ROWS, DH), jnp.bfloat16),
            pltpu.SemaphoreType.DMA,
            pltpu.SemaphoreType.DMA((3,)),
            pltpu.SemaphoreType.DMA((3,)),
            pltpu.SemaphoreType.DMA((3,)),
            pltpu.SemaphoreType.DMA((3,)),
            pltpu.SemaphoreType.DMA((4,)),
            pltpu.SemaphoreType.DMA((4,)),
            pltpu.SemaphoreType.DMA((4,)),
            pltpu.SemaphoreType.DMA((4,)),
            pltpu.SemaphoreType.DMA((8,)),
            pltpu.SemaphoreType.DMA((8,)),
            pltpu.SemaphoreType.DMA((8,)),
            pltpu.SemaphoreType.DMA((8,)),
        ],
        compiler_params=pltpu.CompilerParams(collective_id=0),
    )(x, Wq, kv, Wo)
